# baseline (speedup 1.0000x reference)
import functools

import jax
import jax.numpy as jnp
from jax import lax
from jax.experimental import pallas as pl
from jax.experimental.pallas import tpu as pltpu
from jax.experimental.pallas import tpu_sc as plsc

_B, _CIN, _N, _CO, _K = 8, 3, 2048, 128, 20
_R = 512
_RP = 512
_NT = _N // _RP
_P = _B * _N
_M = _P * _K
_R2 = 512
_NT2 = _P // _R2
_DG = 16
_NC, _NS = 2, 16
_NW = _NC * _NS
_PPT = _P // _NW
_CH = 4
_NCH = _PPT // _CH
_CHI = _CH * _K
_GCH = 128
_GNCH = _M // _NW // _GCH
_NEG = -3.0e38


def _topk_cols(pw, base):
    it = lax.broadcasted_iota(jnp.int32, pw.shape, 1)
    cols, hots = [], []
    for _ in range(_K):
        m = jnp.max(pw, axis=1, keepdims=True)
        j = jnp.min(jnp.where(pw == m, it, _N), axis=1, keepdims=True)
        sel = it == j
        cols.append(j + base)
        hots.append(sel)
        pw = jnp.where(sel, _NEG, pw)
    return cols, hots


def _pairwise(xt, xf):
    dn = (((1,), (0,)), ((), ()))
    inner = lax.dot_general(xt.astype(jnp.bfloat16), xf.astype(jnp.bfloat16),
                            dn, preferred_element_type=jnp.float32)
    nr = jnp.sum(xt * xt, axis=1, keepdims=True)
    nc = jnp.sum(xf * xf, axis=0, keepdims=True)
    return 2.0 * inner - nr - nc


def _prep1_body(xt_ref, x_ref, idx_ref):
    b = pl.program_id(0)
    xt = xt_ref[0]
    xf = x_ref[0]
    pw = _pairwise(xt, xf)
    cols, _ = _topk_cols(pw, b * _N)
    idx_ref[...] = jnp.concatenate(cols, axis=1)


def _prep1(xt3, x3):
    return pl.pallas_call(
        _prep1_body,
        grid=(_B, _NT),
        in_specs=[
            pl.BlockSpec((1, _RP, _DG), lambda b, t: (b, t, 0)),
            pl.BlockSpec((1, _DG, _N), lambda b, t: (b, 0, 0)),
        ],
        out_specs=pl.BlockSpec((_RP, _K), lambda b, t: (b * _NT + t, 0)),
        out_shape=jax.ShapeDtypeStruct((_P, _K), jnp.int32),
    )(xt3, x3)


def _sc_gather_body(tab_hbm, idx_hbm, out_hbm, idx_v, rows_v, sem):
    wid = lax.axis_index("s") * _NC + lax.axis_index("c")
    pltpu.sync_copy(idx_hbm.at[pl.ds(wid * _GNCH, _GNCH)], idx_v)

    def chunk(c, carry):
        base = wid * _GNCH * _GCH + c * _GCH
        pltpu.async_copy(tab_hbm.at[idx_v.at[c]], rows_v, sem).wait()
        pltpu.sync_copy(rows_v, out_hbm.at[pl.ds(base, _GCH)])
        return carry

    lax.fori_loop(0, _GNCH, chunk, 0)


@functools.cache
def _sc_gather_fn():
    return pl.kernel(
        _sc_gather_body,
        mesh=plsc.VectorSubcoreMesh(
            core_axis_name="c", subcore_axis_name="s", num_cores=_NC),
        out_type=jax.ShapeDtypeStruct((_M, _CO), jnp.float32),
        scratch_types=[
            pltpu.VMEM((_GNCH, _GCH), jnp.int32),
            pltpu.VMEM((_GCH, _CO), jnp.float32),
            pltpu.SemaphoreType.DMA,
        ],
    )


def _sc_gather(tab, idx2d):
    return _sc_gather_fn()(tab, idx2d)


def _conv1_body(gat_ref, xi_ref, w_ref, m_ref, sy_ref, sy2_ref, acc):
    ti = pl.program_id(0)

    @pl.when(ti == 0)
    def _init():
        acc[...] = jnp.zeros_like(acc)

    xi = xi_ref[...]
    xib = xi.astype(jnp.bfloat16)
    wb = w_ref[...].astype(jnp.bfloat16)
    dn = (((1,), (0,)), ((), ()))
    m = jnp.full((_R, _CO), _NEG, jnp.float32)
    sy = jnp.zeros((1, _CO), jnp.float32)
    sy2 = jnp.zeros((1, _CO), jnp.float32)
    for k in range(_K):
        d = gat_ref[:, k, :_DG] - xi
        feat = jnp.concatenate([d.astype(jnp.bfloat16), xib], axis=1)
        y = lax.dot_general(feat, wb, dn, preferred_element_type=jnp.float32)
        m = jnp.maximum(m, y)
        sy = sy + jnp.sum(y, axis=0, keepdims=True)
        sy2 = sy2 + jnp.sum(y * y, axis=0, keepdims=True)
    m_ref[...] = m
    acc[0:1] += sy
    acc[1:2] += sy2
    sy_ref[...] = acc[0:1]
    sy2_ref[...] = acc[1:2]


def _conv1(gat3, xpad, w1T):
    return pl.pallas_call(
        _conv1_body,
        grid=(_P // _R,),
        in_specs=[
            pl.BlockSpec((_R, _K, _CO), lambda t: (t, 0, 0)),
            pl.BlockSpec((_R, _DG), lambda t: (t, 0)),
            pl.BlockSpec((2 * _DG, _CO), lambda t: (0, 0)),
        ],
        out_specs=[
            pl.BlockSpec((_R, _CO), lambda t: (t, 0)),
            pl.BlockSpec((1, _CO), lambda t: (0, 0)),
            pl.BlockSpec((1, _CO), lambda t: (0, 0)),
        ],
        out_shape=[
            jax.ShapeDtypeStruct((_P, _CO), jnp.float32),
            jax.ShapeDtypeStruct((1, _CO), jnp.float32),
            jax.ShapeDtypeStruct((1, _CO), jnp.float32),
        ],
        scratch_shapes=[pltpu.VMEM((2, _CO), jnp.float32)],
    )(gat3, xpad, w1T)


def _prep2_body(xt_ref, x_ref, waT_ref, wtT_ref, idx_ref, za_ref, tt_ref):
    b = pl.program_id(0)
    xt = xt_ref[0]
    xf = x_ref[0]
    dn = (((1,), (0,)), ((), ()))
    xtb = xt.astype(jnp.bfloat16)
    za_ref[...] = lax.dot_general(
        xtb, waT_ref[...].astype(jnp.bfloat16), dn,
        preferred_element_type=jnp.float32)
    tt_ref[...] = lax.dot_general(
        xtb, wtT_ref[...].astype(jnp.bfloat16), dn,
        preferred_element_type=jnp.float32)
    pw = _pairwise(xt, xf)
    cols, _ = _topk_cols(pw, b * _N)
    idx_ref[...] = jnp.concatenate(cols, axis=1)


def _prep2(xt3, x3, waT, wtT):
    return pl.pallas_call(
        _prep2_body,
        grid=(_B, _NT),
        in_specs=[
            pl.BlockSpec((1, _RP, _CO), lambda b, t: (b, t, 0)),
            pl.BlockSpec((1, _CO, _N), lambda b, t: (b, 0, 0)),
            pl.BlockSpec((_CO, _CO), lambda b, t: (0, 0)),
            pl.BlockSpec((_CO, _CO), lambda b, t: (0, 0)),
        ],
        out_specs=[
            pl.BlockSpec((_RP, _K), lambda b, t: (b * _NT + t, 0)),
            pl.BlockSpec((_RP, _CO), lambda b, t: (b * _NT + t, 0)),
            pl.BlockSpec((_RP, _CO), lambda b, t: (b * _NT + t, 0)),
        ],
        out_shape=[
            jax.ShapeDtypeStruct((_P, _K), jnp.int32),
            jax.ShapeDtypeStruct((_P, _CO), jnp.float32),
            jax.ShapeDtypeStruct((_P, _CO), jnp.float32),
        ],
    )(xt3, x3, waT, wtT)


def _sc_body(za_hbm, idx_hbm, mx_hbm, s1_hbm, s2_hbm,
             idx_v, rows_v0, rows_v1, om, osum, osq, sem0, sem1):
    wid = lax.axis_index("s") * _NC + lax.axis_index("c")
    pltpu.sync_copy(idx_hbm.at[pl.ds(wid * _NCH, _NCH)], idx_v)

    def reduce_store(rows_v, c):
        base_pt = wid * _PPT + c * _CH
        for p in range(_CH):
            for h in range(_CO // 16):
                sl = pl.ds(h * 16, 16)
                v0 = rows_v[p * _K, sl]

                def jstep(j, acc):
                    am, asm, asq = acc
                    v = rows_v[p * _K + j, sl]
                    return (jnp.maximum(am, v), asm + v, asq + v * v)

                am, asm, asq = lax.fori_loop(1, _K, jstep, (v0, v0, v0 * v0))
                om[p, sl] = am
                osum[p, sl] = asm
                osq[p, sl] = asq
        pltpu.sync_copy(om, mx_hbm.at[pl.ds(base_pt, _CH)])
        pltpu.sync_copy(osum, s1_hbm.at[pl.ds(base_pt, _CH)])
        pltpu.sync_copy(osq, s2_hbm.at[pl.ds(base_pt, _CH)])

    def body(i, carry):
        c0 = 2 * i
        c1 = 2 * i + 1
        pltpu.async_copy(za_hbm.at[idx_v.at[c0]], rows_v0, sem0)
        pltpu.async_copy(za_hbm.at[idx_v.at[c1]], rows_v1, sem1)
        pltpu.make_async_copy(za_hbm.at[idx_v.at[c0]], rows_v0, sem0).wait()
        reduce_store(rows_v0, c0)
        pltpu.make_async_copy(za_hbm.at[idx_v.at[c1]], rows_v1, sem1).wait()
        reduce_store(rows_v1, c1)
        return carry

    lax.fori_loop(0, _NCH // 2, body, 0)


@functools.cache
def _sc_gather_reduce_fn():
    return pl.kernel(
        _sc_body,
        mesh=plsc.VectorSubcoreMesh(
            core_axis_name="c", subcore_axis_name="s", num_cores=_NC),
        out_type=[jax.ShapeDtypeStruct((_P, _CO), jnp.float32)] * 3,
        scratch_types=[
            pltpu.VMEM((_NCH, _CHI), jnp.int32),
            pltpu.VMEM((_CHI, _CO), jnp.float32),
            pltpu.VMEM((_CHI, _CO), jnp.float32),
            pltpu.VMEM((_CH, _CO), jnp.float32),
            pltpu.VMEM((_CH, _CO), jnp.float32),
            pltpu.VMEM((_CH, _CO), jnp.float32),
            pltpu.SemaphoreType.DMA,
            pltpu.SemaphoreType.DMA,
        ],
    )


def _sc_gather_reduce(za, idx2d):
    return _sc_gather_reduce_fn()(za, idx2d)


def _norm1_body(m_ref, sy_ref, sy2_ref, g_ref, bt_ref, o_ref):
    cnt = jnp.float32(_M)
    mean = sy_ref[...] / cnt
    var = sy2_ref[...] / cnt - mean * mean
    inv = lax.rsqrt(var + 1e-5)
    y = (m_ref[...] - mean) * inv * g_ref[...] + bt_ref[...]
    o_ref[...] = jnp.where(y > 0, y, 0.2 * y)


def _normalize1(m, sy, sy2, g, bt):
    row = pl.BlockSpec((_R2, _CO), lambda t: (t, 0))
    vec = pl.BlockSpec((1, _CO), lambda t: (0, 0))
    return pl.pallas_call(
        _norm1_body,
        grid=(_NT2,),
        in_specs=[row, vec, vec, vec, vec],
        out_specs=row,
        out_shape=jax.ShapeDtypeStruct((_P, _CO), jnp.float32),
    )(m, sy, sy2, g.reshape(1, _CO), bt.reshape(1, _CO))


def _norm2_body(mx_ref, s1_ref, s2_ref, tt_ref, g_ref, bt_ref, o_ref, acc):
    ph = pl.program_id(0)
    ti = pl.program_id(1)

    @pl.when(jnp.logical_and(ph == 0, ti == 0))
    def _init():
        acc[...] = jnp.zeros_like(acc)

    @pl.when(ph == 0)
    def _accum():
        s1 = s1_ref[...]
        t = tt_ref[...]
        acc[0:1] += jnp.sum(s1, axis=0, keepdims=True)
        acc[1:2] += jnp.sum(t, axis=0, keepdims=True)
        acc[2:3] += jnp.sum(t * t, axis=0, keepdims=True)
        acc[3:4] += jnp.sum(t * s1, axis=0, keepdims=True)
        acc[4:5] += jnp.sum(s2_ref[...], axis=0, keepdims=True)

    @pl.when(ph == 1)
    def _norm():
        cnt = jnp.float32(_M)
        kf = jnp.float32(_K)
        sumy = acc[0:1] + kf * acc[1:2]
        sumy2 = acc[4:5] + 2.0 * acc[3:4] + kf * acc[2:3]
        mean = sumy / cnt
        var = sumy2 / cnt - mean * mean
        inv = lax.rsqrt(var + 1e-5)
        y = (mx_ref[...] + tt_ref[...] - mean) * inv * g_ref[...] + bt_ref[...]
        o_ref[...] = jnp.where(y > 0, y, 0.2 * y)


def _normalize2(mx, s1, s2, tt, g, bt):
    row = pl.BlockSpec((_R2, _CO), lambda ph, t: (t, 0))
    vec = pl.BlockSpec((1, _CO), lambda ph, t: (0, 0))
    return pl.pallas_call(
        _norm2_body,
        grid=(2, _NT2),
        in_specs=[row, row, row, row, vec, vec],
        out_specs=row,
        out_shape=jax.ShapeDtypeStruct((_P, _CO), jnp.float32),
        scratch_shapes=[pltpu.VMEM((8, _CO), jnp.float32)],
    )(mx, s1, s2, tt, g.reshape(1, _CO), bt.reshape(1, _CO))


def kernel(x, W1, g1, b1, W2, g2, b2):
    return _layer2(_layer1(x, W1, g1, b1), W2, g2, b2)


def _layer1(x, W1, g1, b1):
    xpad3 = jnp.concatenate(
        [x, jnp.zeros((_B, _DG - _CIN, _N), jnp.float32)], axis=1)
    xt1 = jnp.transpose(xpad3, (0, 2, 1))
    idx1 = _prep1(xt1, xpad3)
    xpad = xt1.reshape(_P, _DG)
    tab = jnp.concatenate(
        [xpad, jnp.zeros((_P, _CO - _DG), jnp.float32)], axis=1)
    gat = _sc_gather(tab, idx1.reshape(_M // _GCH, _GCH))
    w1p = jnp.zeros((2 * _DG, _CO), jnp.float32)
    w1p = w1p.at[:_CIN].set(jnp.transpose(W1[:, :_CIN]))
    w1p = w1p.at[_DG:_DG + _CIN].set(jnp.transpose(W1[:, _CIN:]))
    m1, sy1, sy21 = _conv1(gat.reshape(_P, _K, _CO), xpad, w1p)
    return _normalize1(m1, sy1, sy21, g1, b1)


def _layer2(x1t, W2, g2, b2):
    x1_3 = jnp.transpose(x1t.reshape(_B, _N, _CO), (0, 2, 1))
    w2aT = jnp.transpose(W2[:, :_CO])
    w2tT = jnp.transpose(W2[:, _CO:] - W2[:, :_CO])
    idx2, za2, tt2 = _prep2(x1t.reshape(_B, _N, _CO), x1_3, w2aT, w2tT)
    mx2, s12, s22 = _sc_gather_reduce(za2, idx2.reshape(_NW * _NCH, _CHI))
    x2t = _normalize2(mx2, s12, s22, tt2, g2, b2)
    return jnp.transpose(x2t.reshape(_B, _N, _CO), (0, 2, 1))


# speedup vs baseline: 1.0170x; 1.0170x over previous
import functools

import jax
import jax.numpy as jnp
from jax import lax
from jax.experimental import pallas as pl
from jax.experimental.pallas import tpu as pltpu
from jax.experimental.pallas import tpu_sc as plsc

_B, _CIN, _N, _CO, _K = 8, 3, 2048, 128, 20
_R = 512
_RP = 512
_NT = _N // _RP
_P = _B * _N
_M = _P * _K
_R2 = 512
_NT2 = _P // _R2
_DG = 16
_NC, _NS = 2, 16
_NW = _NC * _NS
_PPT = _P // _NW
_CH = 4
_NCH = _PPT // _CH
_CHI = _CH * _K
_GCH = 128
_GNCH = _M // _NW // _GCH
_NEG = -3.0e38


def _topk_cols(pw, base):
    it = lax.broadcasted_iota(jnp.int32, pw.shape, 1)
    cols, hots = [], []
    for _ in range(_K):
        m = jnp.max(pw, axis=1, keepdims=True)
        j = jnp.min(jnp.where(pw == m, it, _N), axis=1, keepdims=True)
        sel = it == j
        cols.append(j + base)
        hots.append(sel)
        pw = jnp.where(sel, _NEG, pw)
    return cols, hots


def _pairwise(xt, xf):
    dn = (((1,), (0,)), ((), ()))
    inner = lax.dot_general(xt.astype(jnp.bfloat16), xf.astype(jnp.bfloat16),
                            dn, preferred_element_type=jnp.float32)
    nr = jnp.sum(xt * xt, axis=1, keepdims=True)
    nc = jnp.sum(xf * xf, axis=0, keepdims=True)
    return 2.0 * inner - nr - nc


def _prep1_body(xt_ref, x_ref, idx_ref):
    b = pl.program_id(0)
    xt = xt_ref[0]
    xf = x_ref[0]
    pw = _pairwise(xt, xf)
    cols, _ = _topk_cols(pw, b * _N)
    idx_ref[...] = jnp.concatenate(cols, axis=1)


def _prep1(xt3, x3):
    return pl.pallas_call(
        _prep1_body,
        grid=(_B, _NT),
        in_specs=[
            pl.BlockSpec((1, _RP, _DG), lambda b, t: (b, t, 0)),
            pl.BlockSpec((1, _DG, _N), lambda b, t: (b, 0, 0)),
        ],
        out_specs=pl.BlockSpec((_RP, _K), lambda b, t: (b * _NT + t, 0)),
        out_shape=jax.ShapeDtypeStruct((_P, _K), jnp.int32),
    )(xt3, x3)


def _sc_gather_body(tab_hbm, idx_hbm, out_hbm, idx_v, rows_v, sem):
    wid = lax.axis_index("s") * _NC + lax.axis_index("c")
    pltpu.sync_copy(idx_hbm.at[pl.ds(wid * _GNCH, _GNCH)], idx_v)

    def chunk(c, carry):
        base = wid * _GNCH * _GCH + c * _GCH
        pltpu.async_copy(tab_hbm.at[idx_v.at[c]], rows_v, sem).wait()
        pltpu.sync_copy(rows_v, out_hbm.at[pl.ds(base, _GCH)])
        return carry

    lax.fori_loop(0, _GNCH, chunk, 0)


@functools.cache
def _sc_gather_fn():
    return pl.kernel(
        _sc_gather_body,
        mesh=plsc.VectorSubcoreMesh(
            core_axis_name="c", subcore_axis_name="s", num_cores=_NC),
        out_type=jax.ShapeDtypeStruct((_M, _CO), jnp.float32),
        scratch_types=[
            pltpu.VMEM((_GNCH, _GCH), jnp.int32),
            pltpu.VMEM((_GCH, _CO), jnp.float32),
            pltpu.SemaphoreType.DMA,
        ],
    )


def _sc_gather(tab, idx2d):
    return _sc_gather_fn()(tab, idx2d)


def _conv1_body(gat_ref, xi_ref, w_ref, m_ref, sy_ref, sy2_ref, acc):
    ti = pl.program_id(0)

    @pl.when(ti == 0)
    def _init():
        acc[...] = jnp.zeros_like(acc)

    xi = xi_ref[...]
    xib = xi.astype(jnp.bfloat16)
    wb = w_ref[...].astype(jnp.bfloat16)
    dn = (((1,), (0,)), ((), ()))
    m = jnp.full((_R, _CO), _NEG, jnp.float32)
    sy = jnp.zeros((1, _CO), jnp.float32)
    sy2 = jnp.zeros((1, _CO), jnp.float32)
    for k in range(_K):
        d = gat_ref[:, k, :_DG] - xi
        feat = jnp.concatenate([d.astype(jnp.bfloat16), xib], axis=1)
        y = lax.dot_general(feat, wb, dn, preferred_element_type=jnp.float32)
        m = jnp.maximum(m, y)
        sy = sy + jnp.sum(y, axis=0, keepdims=True)
        sy2 = sy2 + jnp.sum(y * y, axis=0, keepdims=True)
    m_ref[...] = m
    acc[0:1] += sy
    acc[1:2] += sy2
    sy_ref[...] = acc[0:1]
    sy2_ref[...] = acc[1:2]


def _conv1(gat3, xpad, w1T):
    return pl.pallas_call(
        _conv1_body,
        grid=(_P // _R,),
        in_specs=[
            pl.BlockSpec((_R, _K, _CO), lambda t: (t, 0, 0)),
            pl.BlockSpec((_R, _DG), lambda t: (t, 0)),
            pl.BlockSpec((2 * _DG, _CO), lambda t: (0, 0)),
        ],
        out_specs=[
            pl.BlockSpec((_R, _CO), lambda t: (t, 0)),
            pl.BlockSpec((1, _CO), lambda t: (0, 0)),
            pl.BlockSpec((1, _CO), lambda t: (0, 0)),
        ],
        out_shape=[
            jax.ShapeDtypeStruct((_P, _CO), jnp.float32),
            jax.ShapeDtypeStruct((1, _CO), jnp.float32),
            jax.ShapeDtypeStruct((1, _CO), jnp.float32),
        ],
        scratch_shapes=[pltpu.VMEM((2, _CO), jnp.float32)],
    )(gat3, xpad, w1T)


def _prep2_body(xt_ref, x_ref, waT_ref, wtT_ref, idx_ref, za_ref, tt_ref):
    b = pl.program_id(0)
    xt = xt_ref[0]
    xf = x_ref[0]
    dn = (((1,), (0,)), ((), ()))
    xtb = xt.astype(jnp.bfloat16)
    za_ref[...] = lax.dot_general(
        xtb, waT_ref[...].astype(jnp.bfloat16), dn,
        preferred_element_type=jnp.float32)
    tt_ref[...] = lax.dot_general(
        xtb, wtT_ref[...].astype(jnp.bfloat16), dn,
        preferred_element_type=jnp.float32)
    pw = _pairwise(xt, xf)
    cols, _ = _topk_cols(pw, b * _N)
    idx_ref[...] = jnp.concatenate(cols, axis=1)


def _prep2(xt3, x3, waT, wtT):
    return pl.pallas_call(
        _prep2_body,
        grid=(_B, _NT),
        in_specs=[
            pl.BlockSpec((1, _RP, _CO), lambda b, t: (b, t, 0)),
            pl.BlockSpec((1, _CO, _N), lambda b, t: (b, 0, 0)),
            pl.BlockSpec((_CO, _CO), lambda b, t: (0, 0)),
            pl.BlockSpec((_CO, _CO), lambda b, t: (0, 0)),
        ],
        out_specs=[
            pl.BlockSpec((_RP, _K), lambda b, t: (b * _NT + t, 0)),
            pl.BlockSpec((_RP, _CO), lambda b, t: (b * _NT + t, 0)),
            pl.BlockSpec((_RP, _CO), lambda b, t: (b * _NT + t, 0)),
        ],
        out_shape=[
            jax.ShapeDtypeStruct((_P, _K), jnp.int32),
            jax.ShapeDtypeStruct((_P, _CO), jnp.float32),
            jax.ShapeDtypeStruct((_P, _CO), jnp.float32),
        ],
    )(xt3, x3, waT, wtT)


def _sc_body(za_hbm, idx_hbm, mx_hbm, s1_hbm, s2_hbm,
             idx_v, rows_v0, rows_v1, om, osum, osq, sem0, sem1):
    wid = lax.axis_index("s") * _NC + lax.axis_index("c")
    pltpu.sync_copy(idx_hbm.at[pl.ds(wid * _NCH, _NCH)], idx_v)

    def reduce_store(rows_v, c):
        base_pt = wid * _PPT + c * _CH
        for p in range(_CH):
            for h in range(_CO // 16):
                sl = pl.ds(h * 16, 16)
                v0 = rows_v[p * _K, sl]

                def jstep(j, acc):
                    am, asm, asq = acc
                    v = rows_v[p * _K + j, sl]
                    return (jnp.maximum(am, v), asm + v, asq + v * v)

                am, asm, asq = lax.fori_loop(1, _K, jstep, (v0, v0, v0 * v0))
                om[p, sl] = am
                osum[p, sl] = asm
                osq[p, sl] = asq
        pltpu.sync_copy(om, mx_hbm.at[pl.ds(base_pt, _CH)])
        pltpu.sync_copy(osum, s1_hbm.at[pl.ds(base_pt, _CH)])
        pltpu.sync_copy(osq, s2_hbm.at[pl.ds(base_pt, _CH)])

    def body(i, carry):
        pltpu.async_copy(za_hbm.at[idx_v.at[i]], rows_v0, sem0).wait()
        reduce_store(rows_v0, i)
        return carry

    lax.fori_loop(0, _NCH, body, 0)


@functools.cache
def _sc_gather_reduce_fn():
    return pl.kernel(
        _sc_body,
        mesh=plsc.VectorSubcoreMesh(
            core_axis_name="c", subcore_axis_name="s", num_cores=_NC),
        out_type=[jax.ShapeDtypeStruct((_P, _CO), jnp.float32)] * 3,
        scratch_types=[
            pltpu.VMEM((_NCH, _CHI), jnp.int32),
            pltpu.VMEM((_CHI, _CO), jnp.float32),
            pltpu.VMEM((_CHI, _CO), jnp.float32),
            pltpu.VMEM((_CH, _CO), jnp.float32),
            pltpu.VMEM((_CH, _CO), jnp.float32),
            pltpu.VMEM((_CH, _CO), jnp.float32),
            pltpu.SemaphoreType.DMA,
            pltpu.SemaphoreType.DMA,
        ],
    )


def _sc_gather_reduce(za, idx2d):
    return _sc_gather_reduce_fn()(za, idx2d)


def _norm1_body(m_ref, sy_ref, sy2_ref, g_ref, bt_ref, o_ref):
    cnt = jnp.float32(_M)
    mean = sy_ref[...] / cnt
    var = sy2_ref[...] / cnt - mean * mean
    inv = lax.rsqrt(var + 1e-5)
    y = (m_ref[...] - mean) * inv * g_ref[...] + bt_ref[...]
    o_ref[...] = jnp.where(y > 0, y, 0.2 * y)


def _normalize1(m, sy, sy2, g, bt):
    row = pl.BlockSpec((_R2, _CO), lambda t: (t, 0))
    vec = pl.BlockSpec((1, _CO), lambda t: (0, 0))
    return pl.pallas_call(
        _norm1_body,
        grid=(_NT2,),
        in_specs=[row, vec, vec, vec, vec],
        out_specs=row,
        out_shape=jax.ShapeDtypeStruct((_P, _CO), jnp.float32),
    )(m, sy, sy2, g.reshape(1, _CO), bt.reshape(1, _CO))


def _norm2_body(mx_ref, s1_ref, s2_ref, tt_ref, g_ref, bt_ref, o_ref, acc):
    ph = pl.program_id(0)
    ti = pl.program_id(1)

    @pl.when(jnp.logical_and(ph == 0, ti == 0))
    def _init():
        acc[...] = jnp.zeros_like(acc)

    @pl.when(ph == 0)
    def _accum():
        s1 = s1_ref[...]
        t = tt_ref[...]
        acc[0:1] += jnp.sum(s1, axis=0, keepdims=True)
        acc[1:2] += jnp.sum(t, axis=0, keepdims=True)
        acc[2:3] += jnp.sum(t * t, axis=0, keepdims=True)
        acc[3:4] += jnp.sum(t * s1, axis=0, keepdims=True)
        acc[4:5] += jnp.sum(s2_ref[...], axis=0, keepdims=True)

    @pl.when(ph == 1)
    def _norm():
        cnt = jnp.float32(_M)
        kf = jnp.float32(_K)
        sumy = acc[0:1] + kf * acc[1:2]
        sumy2 = acc[4:5] + 2.0 * acc[3:4] + kf * acc[2:3]
        mean = sumy / cnt
        var = sumy2 / cnt - mean * mean
        inv = lax.rsqrt(var + 1e-5)
        y = (mx_ref[...] + tt_ref[...] - mean) * inv * g_ref[...] + bt_ref[...]
        o_ref[...] = jnp.where(y > 0, y, 0.2 * y)


def _normalize2(mx, s1, s2, tt, g, bt):
    row = pl.BlockSpec((_R2, _CO), lambda ph, t: (t, 0))
    vec = pl.BlockSpec((1, _CO), lambda ph, t: (0, 0))
    return pl.pallas_call(
        _norm2_body,
        grid=(2, _NT2),
        in_specs=[row, row, row, row, vec, vec],
        out_specs=row,
        out_shape=jax.ShapeDtypeStruct((_P, _CO), jnp.float32),
        scratch_shapes=[pltpu.VMEM((8, _CO), jnp.float32)],
    )(mx, s1, s2, tt, g.reshape(1, _CO), bt.reshape(1, _CO))


def kernel(x, W1, g1, b1, W2, g2, b2):
    return _layer2(_layer1(x, W1, g1, b1), W2, g2, b2)


def _layer1(x, W1, g1, b1):
    xpad3 = jnp.concatenate(
        [x, jnp.zeros((_B, _DG - _CIN, _N), jnp.float32)], axis=1)
    xt1 = jnp.transpose(xpad3, (0, 2, 1))
    idx1 = _prep1(xt1, xpad3)
    xpad = xt1.reshape(_P, _DG)
    tab = jnp.concatenate(
        [xpad, jnp.zeros((_P, _CO - _DG), jnp.float32)], axis=1)
    gat = _sc_gather(tab, idx1.reshape(_M // _GCH, _GCH))
    w1p = jnp.zeros((2 * _DG, _CO), jnp.float32)
    w1p = w1p.at[:_CIN].set(jnp.transpose(W1[:, :_CIN]))
    w1p = w1p.at[_DG:_DG + _CIN].set(jnp.transpose(W1[:, _CIN:]))
    m1, sy1, sy21 = _conv1(gat.reshape(_P, _K, _CO), xpad, w1p)
    return _normalize1(m1, sy1, sy21, g1, b1)


def _layer2(x1t, W2, g2, b2):
    x1_3 = jnp.transpose(x1t.reshape(_B, _N, _CO), (0, 2, 1))
    w2aT = jnp.transpose(W2[:, :_CO])
    w2tT = jnp.transpose(W2[:, _CO:] - W2[:, :_CO])
    idx2, za2, tt2 = _prep2(x1t.reshape(_B, _N, _CO), x1_3, w2aT, w2tT)
    mx2, s12, s22 = _sc_gather_reduce(za2, idx2.reshape(_NW * _NCH, _CHI))
    x2t = _normalize2(mx2, s12, s22, tt2, g2, b2)
    return jnp.transpose(x2t.reshape(_B, _N, _CO), (0, 2, 1))


# speedup vs baseline: 1.0746x; 1.0566x over previous
import functools

import jax
import jax.numpy as jnp
from jax import lax
from jax.experimental import pallas as pl
from jax.experimental.pallas import tpu as pltpu
from jax.experimental.pallas import tpu_sc as plsc

_B, _CIN, _N, _CO, _K = 8, 3, 2048, 128, 20
_R = 512
_RP = 512
_NT = _N // _RP
_P = _B * _N
_M = _P * _K
_R2 = 512
_NT2 = _P // _R2
_DG = 16
_NC, _NS = 2, 16
_NW = _NC * _NS
_PPT = _P // _NW
_CH = 4
_NCH = _PPT // _CH
_CHI = _CH * _K
_GCH = 128
_GNCH = _M // _NW // _GCH
_NEG = -3.0e38


def _topk_cols(pw, base):
    it = lax.broadcasted_iota(jnp.int32, pw.shape, 1)
    cols, hots = [], []
    for _ in range(_K):
        m = jnp.max(pw, axis=1, keepdims=True)
        j = jnp.min(jnp.where(pw == m, it, _N), axis=1, keepdims=True)
        sel = it == j
        cols.append(j + base)
        hots.append(sel)
        pw = jnp.where(sel, _NEG, pw)
    return cols, hots


def _pairwise(xt, xf):
    dn = (((1,), (0,)), ((), ()))
    inner = lax.dot_general(xt.astype(jnp.bfloat16), xf.astype(jnp.bfloat16),
                            dn, preferred_element_type=jnp.float32)
    nr = jnp.sum(xt * xt, axis=1, keepdims=True)
    nc = jnp.sum(xf * xf, axis=0, keepdims=True)
    return 2.0 * inner - nr - nc


def _prep1_body(xt_ref, x_ref, idx_ref):
    b = pl.program_id(0)
    xt = xt_ref[0]
    xf = x_ref[0]
    pw = _pairwise(xt, xf)
    cols, _ = _topk_cols(pw, b * _N)
    idx_ref[...] = jnp.concatenate(cols, axis=1)


def _prep1(xt3, x3):
    return pl.pallas_call(
        _prep1_body,
        grid=(_B, _NT),
        in_specs=[
            pl.BlockSpec((1, _RP, _DG), lambda b, t: (b, t, 0)),
            pl.BlockSpec((1, _DG, _N), lambda b, t: (b, 0, 0)),
        ],
        out_specs=pl.BlockSpec((_RP, _K), lambda b, t: (b * _NT + t, 0)),
        out_shape=jax.ShapeDtypeStruct((_P, _K), jnp.int32),
    )(xt3, x3)


def _sc_gather_body(tab_hbm, idx_hbm, out_hbm, idx_v, rows_v, sem):
    wid = lax.axis_index("s") * _NC + lax.axis_index("c")
    pltpu.sync_copy(idx_hbm.at[pl.ds(wid * _GNCH, _GNCH)], idx_v)

    def chunk(c, carry):
        base = wid * _GNCH * _GCH + c * _GCH
        pltpu.async_copy(tab_hbm.at[idx_v.at[c]], rows_v, sem).wait()
        pltpu.sync_copy(rows_v, out_hbm.at[pl.ds(base, _GCH)])
        return carry

    lax.fori_loop(0, _GNCH, chunk, 0)


@functools.cache
def _sc_gather_fn():
    return pl.kernel(
        _sc_gather_body,
        mesh=plsc.VectorSubcoreMesh(
            core_axis_name="c", subcore_axis_name="s", num_cores=_NC),
        out_type=jax.ShapeDtypeStruct((_M, _CO), jnp.float32),
        scratch_types=[
            pltpu.VMEM((_GNCH, _GCH), jnp.int32),
            pltpu.VMEM((_GCH, _CO), jnp.float32),
            pltpu.SemaphoreType.DMA,
        ],
    )


def _sc_gather(tab, idx2d):
    return _sc_gather_fn()(tab, idx2d)


def _conv1_body(gat_ref, xi_ref, w_ref, m_ref, sy_ref, sy2_ref, acc):
    ti = pl.program_id(0)

    @pl.when(ti == 0)
    def _init():
        acc[...] = jnp.zeros_like(acc)

    xi = xi_ref[...]
    xib = xi.astype(jnp.bfloat16)
    wb = w_ref[...].astype(jnp.bfloat16)
    dn = (((1,), (0,)), ((), ()))
    d_all = gat_ref[:, :, :_DG] - xi[:, None, :]
    xib_all = jnp.broadcast_to(xib[:, None, :], (_R, _K, _DG))
    feat = jnp.concatenate([d_all.astype(jnp.bfloat16), xib_all], axis=2)
    y = lax.dot_general(feat.reshape(_R * _K, 2 * _DG), wb, dn,
                        preferred_element_type=jnp.float32)
    sy = jnp.sum(y, axis=0, keepdims=True)
    sy2 = jnp.sum(y * y, axis=0, keepdims=True)
    m = jnp.max(y.reshape(_R, _K, _CO), axis=1)
    m_ref[...] = m
    acc[0:1] += sy
    acc[1:2] += sy2
    sy_ref[...] = acc[0:1]
    sy2_ref[...] = acc[1:2]


def _conv1(gat3, xpad, w1T):
    return pl.pallas_call(
        _conv1_body,
        grid=(_P // _R,),
        in_specs=[
            pl.BlockSpec((_R, _K, _CO), lambda t: (t, 0, 0)),
            pl.BlockSpec((_R, _DG), lambda t: (t, 0)),
            pl.BlockSpec((2 * _DG, _CO), lambda t: (0, 0)),
        ],
        out_specs=[
            pl.BlockSpec((_R, _CO), lambda t: (t, 0)),
            pl.BlockSpec((1, _CO), lambda t: (0, 0)),
            pl.BlockSpec((1, _CO), lambda t: (0, 0)),
        ],
        out_shape=[
            jax.ShapeDtypeStruct((_P, _CO), jnp.float32),
            jax.ShapeDtypeStruct((1, _CO), jnp.float32),
            jax.ShapeDtypeStruct((1, _CO), jnp.float32),
        ],
        scratch_shapes=[pltpu.VMEM((2, _CO), jnp.float32)],
    )(gat3, xpad, w1T)


def _prep2_body(xt_ref, x_ref, waT_ref, wtT_ref, idx_ref, za_ref, tt_ref):
    b = pl.program_id(0)
    xt = xt_ref[0]
    xf = x_ref[0]
    dn = (((1,), (0,)), ((), ()))
    xtb = xt.astype(jnp.bfloat16)
    za_ref[...] = lax.dot_general(
        xtb, waT_ref[...].astype(jnp.bfloat16), dn,
        preferred_element_type=jnp.float32)
    tt_ref[...] = lax.dot_general(
        xtb, wtT_ref[...].astype(jnp.bfloat16), dn,
        preferred_element_type=jnp.float32)
    pw = _pairwise(xt, xf)
    cols, _ = _topk_cols(pw, b * _N)
    idx_ref[...] = jnp.concatenate(cols, axis=1)


def _prep2(xt3, x3, waT, wtT):
    return pl.pallas_call(
        _prep2_body,
        grid=(_B, _NT),
        in_specs=[
            pl.BlockSpec((1, _RP, _CO), lambda b, t: (b, t, 0)),
            pl.BlockSpec((1, _CO, _N), lambda b, t: (b, 0, 0)),
            pl.BlockSpec((_CO, _CO), lambda b, t: (0, 0)),
            pl.BlockSpec((_CO, _CO), lambda b, t: (0, 0)),
        ],
        out_specs=[
            pl.BlockSpec((_RP, _K), lambda b, t: (b * _NT + t, 0)),
            pl.BlockSpec((_RP, _CO), lambda b, t: (b * _NT + t, 0)),
            pl.BlockSpec((_RP, _CO), lambda b, t: (b * _NT + t, 0)),
        ],
        out_shape=[
            jax.ShapeDtypeStruct((_P, _K), jnp.int32),
            jax.ShapeDtypeStruct((_P, _CO), jnp.float32),
            jax.ShapeDtypeStruct((_P, _CO), jnp.float32),
        ],
    )(xt3, x3, waT, wtT)


def _sc_body(za_hbm, idx_hbm, mx_hbm, s1_hbm, s2_hbm,
             idx_v, rows_v0, rows_v1, om, osum, osq, sem0, sem1):
    wid = lax.axis_index("s") * _NC + lax.axis_index("c")
    pltpu.sync_copy(idx_hbm.at[pl.ds(wid * _NCH, _NCH)], idx_v)

    def reduce_store(rows_v, c):
        base_pt = wid * _PPT + c * _CH
        for p in range(_CH):
            for h in range(_CO // 16):
                sl = pl.ds(h * 16, 16)
                v0 = rows_v[p * _K, sl]

                def jstep(j, acc):
                    am, asm, asq = acc
                    v = rows_v[p * _K + j, sl]
                    return (jnp.maximum(am, v), asm + v, asq + v * v)

                am, asm, asq = lax.fori_loop(1, _K, jstep, (v0, v0, v0 * v0))
                om[p, sl] = am
                osum[p, sl] = asm
                osq[p, sl] = asq
        pltpu.sync_copy(om, mx_hbm.at[pl.ds(base_pt, _CH)])
        pltpu.sync_copy(osum, s1_hbm.at[pl.ds(base_pt, _CH)])
        pltpu.sync_copy(osq, s2_hbm.at[pl.ds(base_pt, _CH)])

    def body(i, carry):
        pltpu.async_copy(za_hbm.at[idx_v.at[i]], rows_v0, sem0).wait()
        reduce_store(rows_v0, i)
        return carry

    lax.fori_loop(0, _NCH, body, 0)


@functools.cache
def _sc_gather_reduce_fn():
    return pl.kernel(
        _sc_body,
        mesh=plsc.VectorSubcoreMesh(
            core_axis_name="c", subcore_axis_name="s", num_cores=_NC),
        out_type=[jax.ShapeDtypeStruct((_P, _CO), jnp.float32)] * 3,
        scratch_types=[
            pltpu.VMEM((_NCH, _CHI), jnp.int32),
            pltpu.VMEM((_CHI, _CO), jnp.float32),
            pltpu.VMEM((_CHI, _CO), jnp.float32),
            pltpu.VMEM((_CH, _CO), jnp.float32),
            pltpu.VMEM((_CH, _CO), jnp.float32),
            pltpu.VMEM((_CH, _CO), jnp.float32),
            pltpu.SemaphoreType.DMA,
            pltpu.SemaphoreType.DMA,
        ],
    )


def _sc_gather_reduce(za, idx2d):
    return _sc_gather_reduce_fn()(za, idx2d)


def _norm1_body(m_ref, sy_ref, sy2_ref, g_ref, bt_ref, o_ref):
    cnt = jnp.float32(_M)
    mean = sy_ref[...] / cnt
    var = sy2_ref[...] / cnt - mean * mean
    inv = lax.rsqrt(var + 1e-5)
    y = (m_ref[...] - mean) * inv * g_ref[...] + bt_ref[...]
    o_ref[...] = jnp.where(y > 0, y, 0.2 * y)


def _normalize1(m, sy, sy2, g, bt):
    row = pl.BlockSpec((_R2, _CO), lambda t: (t, 0))
    vec = pl.BlockSpec((1, _CO), lambda t: (0, 0))
    return pl.pallas_call(
        _norm1_body,
        grid=(_NT2,),
        in_specs=[row, vec, vec, vec, vec],
        out_specs=row,
        out_shape=jax.ShapeDtypeStruct((_P, _CO), jnp.float32),
    )(m, sy, sy2, g.reshape(1, _CO), bt.reshape(1, _CO))


def _norm2_body(mx_ref, s1_ref, s2_ref, tt_ref, g_ref, bt_ref, o_ref, acc):
    ph = pl.program_id(0)
    ti = pl.program_id(1)

    @pl.when(jnp.logical_and(ph == 0, ti == 0))
    def _init():
        acc[...] = jnp.zeros_like(acc)

    @pl.when(ph == 0)
    def _accum():
        s1 = s1_ref[...]
        t = tt_ref[...]
        acc[0:1] += jnp.sum(s1, axis=0, keepdims=True)
        acc[1:2] += jnp.sum(t, axis=0, keepdims=True)
        acc[2:3] += jnp.sum(t * t, axis=0, keepdims=True)
        acc[3:4] += jnp.sum(t * s1, axis=0, keepdims=True)
        acc[4:5] += jnp.sum(s2_ref[...], axis=0, keepdims=True)

    @pl.when(ph == 1)
    def _norm():
        cnt = jnp.float32(_M)
        kf = jnp.float32(_K)
        sumy = acc[0:1] + kf * acc[1:2]
        sumy2 = acc[4:5] + 2.0 * acc[3:4] + kf * acc[2:3]
        mean = sumy / cnt
        var = sumy2 / cnt - mean * mean
        inv = lax.rsqrt(var + 1e-5)
        y = (mx_ref[...] + tt_ref[...] - mean) * inv * g_ref[...] + bt_ref[...]
        o_ref[...] = jnp.where(y > 0, y, 0.2 * y)


def _normalize2(mx, s1, s2, tt, g, bt):
    row = pl.BlockSpec((_R2, _CO), lambda ph, t: (t, 0))
    vec = pl.BlockSpec((1, _CO), lambda ph, t: (0, 0))
    return pl.pallas_call(
        _norm2_body,
        grid=(2, _NT2),
        in_specs=[row, row, row, row, vec, vec],
        out_specs=row,
        out_shape=jax.ShapeDtypeStruct((_P, _CO), jnp.float32),
        scratch_shapes=[pltpu.VMEM((8, _CO), jnp.float32)],
    )(mx, s1, s2, tt, g.reshape(1, _CO), bt.reshape(1, _CO))


def kernel(x, W1, g1, b1, W2, g2, b2):
    return _layer2(_layer1(x, W1, g1, b1), W2, g2, b2)


def _layer1(x, W1, g1, b1):
    xpad3 = jnp.concatenate(
        [x, jnp.zeros((_B, _DG - _CIN, _N), jnp.float32)], axis=1)
    xt1 = jnp.transpose(xpad3, (0, 2, 1))
    idx1 = _prep1(xt1, xpad3)
    xpad = xt1.reshape(_P, _DG)
    tab = jnp.concatenate(
        [xpad, jnp.zeros((_P, _CO - _DG), jnp.float32)], axis=1)
    gat = _sc_gather(tab, idx1.reshape(_M // _GCH, _GCH))
    w1p = jnp.zeros((2 * _DG, _CO), jnp.float32)
    w1p = w1p.at[:_CIN].set(jnp.transpose(W1[:, :_CIN]))
    w1p = w1p.at[_DG:_DG + _CIN].set(jnp.transpose(W1[:, _CIN:]))
    m1, sy1, sy21 = _conv1(gat.reshape(_P, _K, _CO), xpad, w1p)
    return _normalize1(m1, sy1, sy21, g1, b1)


def _layer2(x1t, W2, g2, b2):
    x1_3 = jnp.transpose(x1t.reshape(_B, _N, _CO), (0, 2, 1))
    w2aT = jnp.transpose(W2[:, :_CO])
    w2tT = jnp.transpose(W2[:, _CO:] - W2[:, :_CO])
    idx2, za2, tt2 = _prep2(x1t.reshape(_B, _N, _CO), x1_3, w2aT, w2tT)
    mx2, s12, s22 = _sc_gather_reduce(za2, idx2.reshape(_NW * _NCH, _CHI))
    x2t = _normalize2(mx2, s12, s22, tt2, g2, b2)
    return jnp.transpose(x2t.reshape(_B, _N, _CO), (0, 2, 1))


# speedup vs baseline: 1.0751x; 1.0004x over previous
import functools

import jax
import jax.numpy as jnp
from jax import lax
from jax.experimental import pallas as pl
from jax.experimental.pallas import tpu as pltpu
from jax.experimental.pallas import tpu_sc as plsc

_B, _CIN, _N, _CO, _K = 8, 3, 2048, 128, 20
_R = 1024
_RP = 512
_NT = _N // _RP
_P = _B * _N
_M = _P * _K
_R2 = 512
_NT2 = _P // _R2
_DG = 16
_NC, _NS = 2, 16
_NW = _NC * _NS
_PPT = _P // _NW
_CH = 4
_NCH = _PPT // _CH
_CHI = _CH * _K
_GCH = 128
_GNCH = _M // _NW // _GCH
_NEG = -3.0e38


def _topk_cols(pw, base):
    it = lax.broadcasted_iota(jnp.int32, pw.shape, 1)
    cols, hots = [], []
    for _ in range(_K):
        m = jnp.max(pw, axis=1, keepdims=True)
        j = jnp.min(jnp.where(pw == m, it, _N), axis=1, keepdims=True)
        sel = it == j
        cols.append(j + base)
        hots.append(sel)
        pw = jnp.where(sel, _NEG, pw)
    return cols, hots


def _pairwise(xt, xf):
    dn = (((1,), (0,)), ((), ()))
    inner = lax.dot_general(xt.astype(jnp.bfloat16), xf.astype(jnp.bfloat16),
                            dn, preferred_element_type=jnp.float32)
    nr = jnp.sum(xt * xt, axis=1, keepdims=True)
    nc = jnp.sum(xf * xf, axis=0, keepdims=True)
    return 2.0 * inner - nr - nc


def _prep1_body(xt_ref, x_ref, idx_ref):
    b = pl.program_id(0)
    xt = xt_ref[0]
    xf = x_ref[0]
    pw = _pairwise(xt, xf)
    cols, _ = _topk_cols(pw, b * _N)
    idx_ref[...] = jnp.concatenate(cols, axis=1)


def _prep1(xt3, x3):
    return pl.pallas_call(
        _prep1_body,
        grid=(_B, _NT),
        in_specs=[
            pl.BlockSpec((1, _RP, _DG), lambda b, t: (b, t, 0)),
            pl.BlockSpec((1, _DG, _N), lambda b, t: (b, 0, 0)),
        ],
        out_specs=pl.BlockSpec((_RP, _K), lambda b, t: (b * _NT + t, 0)),
        out_shape=jax.ShapeDtypeStruct((_P, _K), jnp.int32),
    )(xt3, x3)


def _sc_gather_body(tab_hbm, idx_hbm, out_hbm, idx_v, rows_v, sem):
    wid = lax.axis_index("s") * _NC + lax.axis_index("c")
    pltpu.sync_copy(idx_hbm.at[pl.ds(wid * _GNCH, _GNCH)], idx_v)

    def chunk(c, carry):
        base = wid * _GNCH * _GCH + c * _GCH
        pltpu.async_copy(tab_hbm.at[idx_v.at[c]], rows_v, sem).wait()
        pltpu.sync_copy(rows_v, out_hbm.at[pl.ds(base, _GCH)])
        return carry

    lax.fori_loop(0, _GNCH, chunk, 0)


@functools.cache
def _sc_gather_fn():
    return pl.kernel(
        _sc_gather_body,
        mesh=plsc.VectorSubcoreMesh(
            core_axis_name="c", subcore_axis_name="s", num_cores=_NC),
        out_type=jax.ShapeDtypeStruct((_M, _CO), jnp.float32),
        scratch_types=[
            pltpu.VMEM((_GNCH, _GCH), jnp.int32),
            pltpu.VMEM((_GCH, _CO), jnp.float32),
            pltpu.SemaphoreType.DMA,
        ],
    )


def _sc_gather(tab, idx2d):
    return _sc_gather_fn()(tab, idx2d)


def _conv1_body(gat_ref, xi_ref, w_ref, m_ref, sy_ref, sy2_ref, acc):
    ti = pl.program_id(0)

    @pl.when(ti == 0)
    def _init():
        acc[...] = jnp.zeros_like(acc)

    xi = xi_ref[...]
    xib = xi.astype(jnp.bfloat16)
    wb = w_ref[...].astype(jnp.bfloat16)
    dn = (((1,), (0,)), ((), ()))
    d_all = gat_ref[:, :, :_DG] - xi[:, None, :]
    xib_all = jnp.broadcast_to(xib[:, None, :], (_R, _K, _DG))
    feat = jnp.concatenate([d_all.astype(jnp.bfloat16), xib_all], axis=2)
    y = lax.dot_general(feat.reshape(_R * _K, 2 * _DG), wb, dn,
                        preferred_element_type=jnp.float32)
    sy = jnp.sum(y, axis=0, keepdims=True)
    sy2 = jnp.sum(y * y, axis=0, keepdims=True)
    m = jnp.max(y.reshape(_R, _K, _CO), axis=1)
    m_ref[...] = m
    acc[0:1] += sy
    acc[1:2] += sy2
    sy_ref[...] = acc[0:1]
    sy2_ref[...] = acc[1:2]


def _conv1(gat3, xpad, w1T):
    return pl.pallas_call(
        _conv1_body,
        grid=(_P // _R,),
        in_specs=[
            pl.BlockSpec((_R, _K, _CO), lambda t: (t, 0, 0)),
            pl.BlockSpec((_R, _DG), lambda t: (t, 0)),
            pl.BlockSpec((2 * _DG, _CO), lambda t: (0, 0)),
        ],
        out_specs=[
            pl.BlockSpec((_R, _CO), lambda t: (t, 0)),
            pl.BlockSpec((1, _CO), lambda t: (0, 0)),
            pl.BlockSpec((1, _CO), lambda t: (0, 0)),
        ],
        out_shape=[
            jax.ShapeDtypeStruct((_P, _CO), jnp.float32),
            jax.ShapeDtypeStruct((1, _CO), jnp.float32),
            jax.ShapeDtypeStruct((1, _CO), jnp.float32),
        ],
        scratch_shapes=[pltpu.VMEM((2, _CO), jnp.float32)],
    )(gat3, xpad, w1T)


def _prep2_body(xt_ref, x_ref, waT_ref, wtT_ref, idx_ref, za_ref, tt_ref):
    b = pl.program_id(0)
    xt = xt_ref[0]
    xf = x_ref[0]
    dn = (((1,), (0,)), ((), ()))
    xtb = xt.astype(jnp.bfloat16)
    za_ref[...] = lax.dot_general(
        xtb, waT_ref[...].astype(jnp.bfloat16), dn,
        preferred_element_type=jnp.float32)
    tt_ref[...] = lax.dot_general(
        xtb, wtT_ref[...].astype(jnp.bfloat16), dn,
        preferred_element_type=jnp.float32)
    pw = _pairwise(xt, xf)
    cols, _ = _topk_cols(pw, b * _N)
    idx_ref[...] = jnp.concatenate(cols, axis=1)


def _prep2(xt3, x3, waT, wtT):
    return pl.pallas_call(
        _prep2_body,
        grid=(_B, _NT),
        in_specs=[
            pl.BlockSpec((1, _RP, _CO), lambda b, t: (b, t, 0)),
            pl.BlockSpec((1, _CO, _N), lambda b, t: (b, 0, 0)),
            pl.BlockSpec((_CO, _CO), lambda b, t: (0, 0)),
            pl.BlockSpec((_CO, _CO), lambda b, t: (0, 0)),
        ],
        out_specs=[
            pl.BlockSpec((_RP, _K), lambda b, t: (b * _NT + t, 0)),
            pl.BlockSpec((_RP, _CO), lambda b, t: (b * _NT + t, 0)),
            pl.BlockSpec((_RP, _CO), lambda b, t: (b * _NT + t, 0)),
        ],
        out_shape=[
            jax.ShapeDtypeStruct((_P, _K), jnp.int32),
            jax.ShapeDtypeStruct((_P, _CO), jnp.float32),
            jax.ShapeDtypeStruct((_P, _CO), jnp.float32),
        ],
    )(xt3, x3, waT, wtT)


def _sc_body(za_hbm, idx_hbm, mx_hbm, s1_hbm, s2_hbm,
             idx_v, rows_v0, rows_v1, om, osum, osq, sem0, sem1):
    wid = lax.axis_index("s") * _NC + lax.axis_index("c")
    pltpu.sync_copy(idx_hbm.at[pl.ds(wid * _NCH, _NCH)], idx_v)

    def reduce_store(rows_v, c):
        base_pt = wid * _PPT + c * _CH
        for p in range(_CH):
            for h in range(_CO // 16):
                sl = pl.ds(h * 16, 16)
                v0 = rows_v[p * _K, sl]

                def jstep(j, acc):
                    am, asm, asq = acc
                    v = rows_v[p * _K + j, sl]
                    return (jnp.maximum(am, v), asm + v, asq + v * v)

                am, asm, asq = lax.fori_loop(1, _K, jstep, (v0, v0, v0 * v0))
                om[p, sl] = am
                osum[p, sl] = asm
                osq[p, sl] = asq
        pltpu.sync_copy(om, mx_hbm.at[pl.ds(base_pt, _CH)])
        pltpu.sync_copy(osum, s1_hbm.at[pl.ds(base_pt, _CH)])
        pltpu.sync_copy(osq, s2_hbm.at[pl.ds(base_pt, _CH)])

    def body(i, carry):
        pltpu.async_copy(za_hbm.at[idx_v.at[i]], rows_v0, sem0).wait()
        reduce_store(rows_v0, i)
        return carry

    lax.fori_loop(0, _NCH, body, 0)


@functools.cache
def _sc_gather_reduce_fn():
    return pl.kernel(
        _sc_body,
        mesh=plsc.VectorSubcoreMesh(
            core_axis_name="c", subcore_axis_name="s", num_cores=_NC),
        out_type=[jax.ShapeDtypeStruct((_P, _CO), jnp.float32)] * 3,
        scratch_types=[
            pltpu.VMEM((_NCH, _CHI), jnp.int32),
            pltpu.VMEM((_CHI, _CO), jnp.float32),
            pltpu.VMEM((_CHI, _CO), jnp.float32),
            pltpu.VMEM((_CH, _CO), jnp.float32),
            pltpu.VMEM((_CH, _CO), jnp.float32),
            pltpu.VMEM((_CH, _CO), jnp.float32),
            pltpu.SemaphoreType.DMA,
            pltpu.SemaphoreType.DMA,
        ],
    )


def _sc_gather_reduce(za, idx2d):
    return _sc_gather_reduce_fn()(za, idx2d)


def _norm1_body(m_ref, sy_ref, sy2_ref, g_ref, bt_ref, o_ref):
    cnt = jnp.float32(_M)
    mean = sy_ref[...] / cnt
    var = sy2_ref[...] / cnt - mean * mean
    inv = lax.rsqrt(var + 1e-5)
    y = (m_ref[...] - mean) * inv * g_ref[...] + bt_ref[...]
    o_ref[...] = jnp.where(y > 0, y, 0.2 * y)


def _normalize1(m, sy, sy2, g, bt):
    row = pl.BlockSpec((_R2, _CO), lambda t: (t, 0))
    vec = pl.BlockSpec((1, _CO), lambda t: (0, 0))
    return pl.pallas_call(
        _norm1_body,
        grid=(_NT2,),
        in_specs=[row, vec, vec, vec, vec],
        out_specs=row,
        out_shape=jax.ShapeDtypeStruct((_P, _CO), jnp.float32),
    )(m, sy, sy2, g.reshape(1, _CO), bt.reshape(1, _CO))


def _norm2_body(mx_ref, s1_ref, s2_ref, tt_ref, g_ref, bt_ref, o_ref, acc):
    ph = pl.program_id(0)
    ti = pl.program_id(1)

    @pl.when(jnp.logical_and(ph == 0, ti == 0))
    def _init():
        acc[...] = jnp.zeros_like(acc)

    @pl.when(ph == 0)
    def _accum():
        s1 = s1_ref[...]
        t = tt_ref[...]
        acc[0:1] += jnp.sum(s1, axis=0, keepdims=True)
        acc[1:2] += jnp.sum(t, axis=0, keepdims=True)
        acc[2:3] += jnp.sum(t * t, axis=0, keepdims=True)
        acc[3:4] += jnp.sum(t * s1, axis=0, keepdims=True)
        acc[4:5] += jnp.sum(s2_ref[...], axis=0, keepdims=True)

    @pl.when(ph == 1)
    def _norm():
        cnt = jnp.float32(_M)
        kf = jnp.float32(_K)
        sumy = acc[0:1] + kf * acc[1:2]
        sumy2 = acc[4:5] + 2.0 * acc[3:4] + kf * acc[2:3]
        mean = sumy / cnt
        var = sumy2 / cnt - mean * mean
        inv = lax.rsqrt(var + 1e-5)
        y = (mx_ref[...] + tt_ref[...] - mean) * inv * g_ref[...] + bt_ref[...]
        o_ref[...] = jnp.where(y > 0, y, 0.2 * y)


def _normalize2(mx, s1, s2, tt, g, bt):
    row = pl.BlockSpec((_R2, _CO), lambda ph, t: (t, 0))
    vec = pl.BlockSpec((1, _CO), lambda ph, t: (0, 0))
    return pl.pallas_call(
        _norm2_body,
        grid=(2, _NT2),
        in_specs=[row, row, row, row, vec, vec],
        out_specs=row,
        out_shape=jax.ShapeDtypeStruct((_P, _CO), jnp.float32),
        scratch_shapes=[pltpu.VMEM((8, _CO), jnp.float32)],
    )(mx, s1, s2, tt, g.reshape(1, _CO), bt.reshape(1, _CO))


def kernel(x, W1, g1, b1, W2, g2, b2):
    return _layer2(_layer1(x, W1, g1, b1), W2, g2, b2)


def _layer1(x, W1, g1, b1):
    xpad3 = jnp.concatenate(
        [x, jnp.zeros((_B, _DG - _CIN, _N), jnp.float32)], axis=1)
    xt1 = jnp.transpose(xpad3, (0, 2, 1))
    idx1 = _prep1(xt1, xpad3)
    xpad = xt1.reshape(_P, _DG)
    tab = jnp.concatenate(
        [xpad, jnp.zeros((_P, _CO - _DG), jnp.float32)], axis=1)
    gat = _sc_gather(tab, idx1.reshape(_M // _GCH, _GCH))
    w1p = jnp.zeros((2 * _DG, _CO), jnp.float32)
    w1p = w1p.at[:_CIN].set(jnp.transpose(W1[:, :_CIN]))
    w1p = w1p.at[_DG:_DG + _CIN].set(jnp.transpose(W1[:, _CIN:]))
    m1, sy1, sy21 = _conv1(gat.reshape(_P, _K, _CO), xpad, w1p)
    return _normalize1(m1, sy1, sy21, g1, b1)


def _layer2(x1t, W2, g2, b2):
    x1_3 = jnp.transpose(x1t.reshape(_B, _N, _CO), (0, 2, 1))
    w2aT = jnp.transpose(W2[:, :_CO])
    w2tT = jnp.transpose(W2[:, _CO:] - W2[:, :_CO])
    idx2, za2, tt2 = _prep2(x1t.reshape(_B, _N, _CO), x1_3, w2aT, w2tT)
    mx2, s12, s22 = _sc_gather_reduce(za2, idx2.reshape(_NW * _NCH, _CHI))
    x2t = _normalize2(mx2, s12, s22, tt2, g2, b2)
    return jnp.transpose(x2t.reshape(_B, _N, _CO), (0, 2, 1))
